# baseline (speedup 1.0000x reference)
import functools

import jax
import jax.numpy as jnp
from jax import lax
from jax.experimental import pallas as pl

B, N, H = 8, 64, 256
NE, DE = 64, 32
RBF_DIM = 64
CUTOFF = 6.0
PMAX = 256
MAX_Z = 100
Z_EMB = 64
SCATTER = 64
GEOM_HID = 512
PAIR_HID = 256
OUT_DIM = 256
ZPAD = 128
EPAD = 128
ECHUNK = 8


def _silu_h(hx):
    return hx + hx * jnp.tanh(hx)


def _body(pos_ref, maskf_ref, zone_ref, h_ref, const_ref, ef_ref, zemb_ref,
          w1hj_ref, w1hk_ref, w1fj_ref, w1fk_ref, w1jk_ref, w1c_ref, b1_ref,
          w2_ref, b2_ref, w3_ref, b3_ref,
          pj_ref, pk_ref, pe_ref, pb1_ref, p2_ref, pb2_ref, p3_ref, pb3_ref,
          o1_ref, ob1_ref, o2_ref, ob2_ref, out_ref):
    f32 = jnp.float32

    pos = pos_ref[0]
    rel = pos - pos[0:1, :]
    r0sq = jnp.sum(rel * rel, axis=1, keepdims=True)
    r0 = jnp.sqrt(r0sq)
    aidx = lax.broadcasted_iota(jnp.int32, (N, 1), 0).astype(f32)
    validc = jnp.where(
        (maskf_ref[0][:, 0:1] > 0.0) & (r0 <= CUTOFF) & (aidx > 0.0),
        1.0, 0.0).astype(f32)

    eye = const_ref[0]
    triu = const_ref[1]
    ltinc = const_ref[2]
    slow = const_ref[3]
    ones_nn = jnp.ones((N, N), f32)

    vcb = jnp.broadcast_to(validc, (N, N))
    vrow = jnp.dot(ones_nn, vcb * eye, preferred_element_type=f32)
    pv = vcb * vrow * triu

    cinc = jnp.dot(pv, ltinc, preferred_element_type=f32)
    rcnt = cinc[:, N - 1:N]
    rcb = jnp.broadcast_to(rcnt, (N, N))
    offm = jnp.dot(slow, rcb, preferred_element_type=f32)

    ones_sn = jnp.ones((PMAX, N), f32)
    O = jnp.dot(ones_sn, offm * eye, preferred_element_type=f32)
    Rc = jnp.dot(ones_sn, rcb * eye, preferred_element_type=f32)
    siota = lax.broadcasted_iota(jnp.int32, (PMAX, N), 0).astype(f32)
    Jone = jnp.where((siota >= O) & (siota < O + Rc), 1.0, 0.0).astype(f32)
    pmask = jnp.sum(Jone, axis=1, keepdims=True)

    oslot = jnp.sum(Jone * O, axis=1, keepdims=True)
    scol = lax.broadcasted_iota(jnp.int32, (PMAX, 1), 0).astype(f32)
    t = scol - oslot
    wm = cinc * pv - 1.0
    wslot = jnp.dot(Jone, wm, preferred_element_type=f32)
    Kone = jnp.where(jnp.abs(wslot - t) < 0.5, 1.0, 0.0).astype(f32) * pmask

    Vj = jnp.dot(Jone, rel, preferred_element_type=f32)
    Vk = jnp.dot(Kone, rel, preferred_element_type=f32)
    r0j = jnp.sqrt(jnp.sum(Vj * Vj, axis=1, keepdims=True))
    r0k = jnp.sqrt(jnp.sum(Vk * Vk, axis=1, keepdims=True))
    vjk = Vk - Vj
    rjk = jnp.sqrt(jnp.sum(vjk * vjk, axis=1, keepdims=True))
    dotjk = jnp.sum(Vj * Vk, axis=1, keepdims=True)
    cosang = jnp.clip(
        dotjk / (jnp.maximum(r0j, 1e-8) * jnp.maximum(r0k, 1e-8)), -1.0, 1.0)

    delta = CUTOFF / (RBF_DIM - 1)
    gamma = 1.0 / (delta * delta + 1e-12)
    centers = lax.broadcasted_iota(jnp.int32, (1, RBF_DIM), 1).astype(f32) * delta

    d_at = jnp.minimum(r0, CUTOFF) - centers
    rbf_at = jnp.exp(-gamma * d_at * d_at)
    d_jk = jnp.minimum(rjk, CUTOFF) - centers
    fjk = jnp.exp(-gamma * d_jk * d_jk)

    hb = h_ref[0]
    atomj = (jnp.dot(hb, w1hj_ref[...], preferred_element_type=f32)
             + jnp.dot(rbf_at, w1fj_ref[...], preferred_element_type=f32))
    atomk = (jnp.dot(hb, w1hk_ref[...], preferred_element_type=f32)
             + jnp.dot(rbf_at, w1fk_ref[...], preferred_element_type=f32))
    pre = (jnp.dot(Jone, atomj, preferred_element_type=f32)
           + jnp.dot(Kone, atomk, preferred_element_type=f32)
           + jnp.dot(fjk, w1jk_ref[...], preferred_element_type=f32)
           + cosang * w1c_ref[...]
           + b1_ref[...])
    x = _silu_h(pre)
    x = _silu_h(jnp.dot(x, w2_ref[...], preferred_element_type=f32) + b2_ref[...])
    gg = jnp.dot(x, w3_ref[...], preferred_element_type=f32) + b3_ref[...]
    ggm = gg * pmask

    ze = jnp.dot(zone_ref[0], zemb_ref[...], preferred_element_type=f32)
    apj = jnp.dot(ze, pj_ref[...], preferred_element_type=f32)
    apk = jnp.dot(ze, pk_ref[...], preferred_element_type=f32)
    ce = jnp.dot(ef_ref[...], pe_ref[...], preferred_element_type=f32)
    base = (jnp.dot(Jone, apj, preferred_element_type=f32)
            + jnp.dot(Kone, apk, preferred_element_type=f32)
            + pb1_ref[...])

    bf16 = jnp.bfloat16
    base_b = base.astype(bf16)
    ce_b = ce.astype(bf16)
    p2 = p2_ref[...].astype(bf16)
    pb2 = pb2_ref[...].astype(bf16)
    p3 = p3_ref[...].astype(bf16)
    pb3 = pb3_ref[...]

    agg_chunks = []
    for ec in range(NE // ECHUNK):
        y1 = jnp.concatenate(
            [_silu_h(base_b + ce_b[ec * ECHUNK + i:ec * ECHUNK + i + 1, :])
             for i in range(ECHUNK)], axis=0)
        y2 = _silu_h(jnp.dot(y1, p2,
                             preferred_element_type=f32).astype(bf16) + pb2)
        ge = jnp.dot(y2, p3, preferred_element_type=f32) + pb3
        contrib = ge.reshape(ECHUNK, PMAX, SCATTER) * ggm[None, :, :]
        agg_chunks.append(jnp.sum(contrib, axis=1))
    agg = jnp.concatenate(agg_chunks, axis=0)

    oo = _silu_h(jnp.dot(agg, o1_ref[...], preferred_element_type=f32)
               + ob1_ref[...])
    out_ref[0] = jnp.dot(oo, o2_ref[...], preferred_element_type=f32) + ob2_ref[...]


@jax.jit
def kernel(h, z, pos, mask, e_feat, z_emb, gw1, gb1, gw2, gb2, gw3, gb3,
           pw1, pb1, pw2, pb2, pw3, pb3, ow1, ob1, ow2, ob2):
    f32 = jnp.float32
    h = h.astype(f32)
    pos_pad = jnp.pad(pos.astype(f32), ((0, 0), (0, 0), (0, 5)))
    maskf = jnp.broadcast_to(mask.astype(f32)[:, :, None], (B, N, 8))
    zone = jax.nn.one_hot(z, ZPAD, dtype=f32)
    ef_pad = jnp.pad(e_feat.astype(f32), ((0, 0), (0, EPAD - DE)))
    zemb_pad = jnp.pad(z_emb.astype(f32), ((0, ZPAD - (MAX_Z + 1)), (0, 0)))

    gw1h = 0.5 * gw1
    w1hj = gw1h[0:H]
    w1hk = gw1h[H:2 * H]
    w1fj = gw1h[2 * H:2 * H + RBF_DIM]
    w1fk = gw1h[2 * H + RBF_DIM:2 * H + 2 * RBF_DIM]
    w1jk = gw1h[2 * H + 2 * RBF_DIM:2 * H + 3 * RBF_DIM]
    w1c = gw1h[2 * H + 3 * RBF_DIM:]
    pw1h = 0.5 * pw1
    pj = pw1h[0:Z_EMB]
    pk = pw1h[Z_EMB:2 * Z_EMB]
    pe = jnp.pad(pw1h[2 * Z_EMB:], ((0, EPAD - DE), (0, 0)))
    gb1 = 0.5 * gb1
    gw2, gb2 = 0.5 * gw2, 0.5 * gb2
    pb1 = 0.5 * pb1
    pw2, pb2 = 0.5 * pw2, 0.5 * pb2
    ow1, ob1 = 0.5 * ow1, 0.5 * ob1

    row = lambda v: v.reshape(1, -1)

    def bspec(shape):
        nd = len(shape)
        return pl.BlockSpec((1,) + shape[1:], lambda b, _n=nd: (b,) + (0,) * (_n - 1))

    def wspec(shape):
        nd = len(shape)
        return pl.BlockSpec(shape, lambda b, _n=nd: (0,) * _n)

    ii = jnp.arange(N)
    consts = jnp.stack([
        (ii[:, None] == ii[None, :]).astype(f32),
        (ii[:, None] < ii[None, :]).astype(f32),
        (ii[:, None] <= ii[None, :]).astype(f32),
        (ii[None, :] < ii[:, None]).astype(f32),
    ])

    batch_in = [pos_pad, maskf, zone, h]
    weights = [consts, ef_pad, zemb_pad,
               w1hj, w1hk, w1fj, w1fk, w1jk, w1c, row(gb1),
               gw2, row(gb2), gw3, row(gb3),
               pj, pk, pe, row(pb1), pw2, row(pb2), pw3, row(pb3),
               ow1, row(ob1), ow2, row(ob2)]

    out = pl.pallas_call(
        _body,
        grid=(B,),
        in_specs=[bspec(a.shape) for a in batch_in]
                 + [wspec(w.shape) for w in weights],
        out_specs=bspec((B, NE, OUT_DIM)),
        out_shape=jax.ShapeDtypeStruct((B, NE, OUT_DIM), f32),
    )(*batch_in, *weights)
    return out


# speedup vs baseline: 1.0879x; 1.0879x over previous
import jax
import jax.numpy as jnp
from jax import lax
from jax.experimental import pallas as pl

B, N, H = 8, 64, 256
NE, DE = 64, 32
RBF_DIM = 64
CUTOFF = 6.0
PMAX = 256
MAX_Z = 100
Z_EMB = 64
SCATTER = 64
GEOM_HID = 512
PAIR_HID = 256
OUT_DIM = 256
ZPAD = 128
ECHUNK = 8


def _silu_h(hx):
    return hx + hx * jnp.tanh(hx)


def _body(pos_ref, zone_ref, h_ref, const_ref, ef_ref, zemb_ref,
          gw1_ref, b1_ref, w2_ref, b2_ref, w3_ref, b3_ref,
          pw1_ref, pb1_ref, p2_ref, pb2_ref, p3_ref, pb3_ref,
          o1_ref, ob1_ref, o2_ref, ob2_ref, out_ref):
    f32 = jnp.float32

    posm = pos_ref[0]
    lidx = lax.broadcasted_iota(jnp.int32, (1, 8), 1)
    xyz = jnp.where(lidx < 3, posm, 0.0)
    rel = xyz - xyz[0:1, :]
    r0sq = jnp.sum(rel * rel, axis=1, keepdims=True)
    r0 = jnp.sqrt(r0sq)
    aidx = lax.broadcasted_iota(jnp.int32, (N, 1), 0).astype(f32)
    validc = jnp.where(
        (posm[:, 3:4] > 0.0) & (r0 <= CUTOFF) & (aidx > 0.0),
        1.0, 0.0).astype(f32)

    eye = const_ref[0]
    triu = const_ref[1]
    ltinc = const_ref[2]
    slow = const_ref[3]
    ones_nn = jnp.ones((N, N), f32)

    vcb = jnp.broadcast_to(validc, (N, N))
    vrow = jnp.dot(ones_nn, vcb * eye, preferred_element_type=f32)
    pv = vcb * vrow * triu

    cinc = jnp.dot(pv, ltinc, preferred_element_type=f32)
    rcnt = cinc[:, N - 1:N]
    rcb = jnp.broadcast_to(rcnt, (N, N))
    offm = jnp.dot(slow, rcb, preferred_element_type=f32)

    ones_sn = jnp.ones((PMAX, N), f32)
    O = jnp.dot(ones_sn, offm * eye, preferred_element_type=f32)
    Rc = jnp.dot(ones_sn, rcb * eye, preferred_element_type=f32)
    siota = lax.broadcasted_iota(jnp.int32, (PMAX, N), 0).astype(f32)
    Jone = jnp.where((siota >= O) & (siota < O + Rc), 1.0, 0.0).astype(f32)
    pmask = jnp.sum(Jone, axis=1, keepdims=True)

    oslot = jnp.sum(Jone * O, axis=1, keepdims=True)
    scol = lax.broadcasted_iota(jnp.int32, (PMAX, 1), 0).astype(f32)
    t = scol - oslot
    wm = cinc * pv - 1.0
    wslot = jnp.dot(Jone, wm, preferred_element_type=f32)
    Kone = jnp.where(jnp.abs(wslot - t) < 0.5, 1.0, 0.0).astype(f32) * pmask

    Vj = jnp.dot(Jone, rel, preferred_element_type=f32)
    Vk = jnp.dot(Kone, rel, preferred_element_type=f32)
    r0j = jnp.sqrt(jnp.sum(Vj * Vj, axis=1, keepdims=True))
    r0k = jnp.sqrt(jnp.sum(Vk * Vk, axis=1, keepdims=True))
    vjk = Vk - Vj
    rjk = jnp.sqrt(jnp.sum(vjk * vjk, axis=1, keepdims=True))
    dotjk = jnp.sum(Vj * Vk, axis=1, keepdims=True)
    cosang = jnp.clip(
        dotjk / (jnp.maximum(r0j, 1e-8) * jnp.maximum(r0k, 1e-8)), -1.0, 1.0)

    delta = CUTOFF / (RBF_DIM - 1)
    gamma = 1.0 / (delta * delta + 1e-12)
    centers = lax.broadcasted_iota(jnp.int32, (1, RBF_DIM), 1).astype(f32) * delta

    d_at = jnp.minimum(r0, CUTOFF) - centers
    rbf_at = jnp.exp(-gamma * d_at * d_at)
    d_jk = jnp.minimum(rjk, CUTOFF) - centers
    fjk = jnp.exp(-gamma * d_jk * d_jk)

    hb = h_ref[0]
    w1hj = gw1_ref[0:H, :]
    w1hk = gw1_ref[H:2 * H, :]
    w1fj = gw1_ref[2 * H:2 * H + RBF_DIM, :]
    w1fk = gw1_ref[2 * H + RBF_DIM:2 * H + 2 * RBF_DIM, :]
    w1jk = gw1_ref[2 * H + 2 * RBF_DIM:2 * H + 3 * RBF_DIM, :]
    w1c = gw1_ref[2 * H + 3 * RBF_DIM:2 * H + 3 * RBF_DIM + 1, :]
    atomj = (jnp.dot(hb, w1hj, preferred_element_type=f32)
             + jnp.dot(rbf_at, w1fj, preferred_element_type=f32))
    atomk = (jnp.dot(hb, w1hk, preferred_element_type=f32)
             + jnp.dot(rbf_at, w1fk, preferred_element_type=f32))
    pre = (jnp.dot(Jone, atomj, preferred_element_type=f32)
           + jnp.dot(Kone, atomk, preferred_element_type=f32)
           + jnp.dot(fjk, w1jk, preferred_element_type=f32)
           + cosang * w1c
           + b1_ref[...])
    x = _silu_h(0.5 * pre)
    x = _silu_h(0.5 * (jnp.dot(x, w2_ref[...], preferred_element_type=f32)
                       + b2_ref[...]))
    gg = jnp.dot(x, w3_ref[...], preferred_element_type=f32) + b3_ref[...]
    ggm = gg * pmask

    ze = jnp.dot(zone_ref[0], zemb_ref[...], preferred_element_type=f32)
    pj = pw1_ref[0:Z_EMB, :]
    pk = pw1_ref[Z_EMB:2 * Z_EMB, :]
    pe = pw1_ref[2 * Z_EMB:2 * Z_EMB + DE, :]
    apj = jnp.dot(ze, pj, preferred_element_type=f32)
    apk = jnp.dot(ze, pk, preferred_element_type=f32)
    ce = jnp.dot(ef_ref[...], pe, preferred_element_type=f32)
    base = (jnp.dot(Jone, apj, preferred_element_type=f32)
            + jnp.dot(Kone, apk, preferred_element_type=f32)
            + pb1_ref[...])

    bf16 = jnp.bfloat16
    base_b = (0.5 * base).astype(bf16)
    ce_b = (0.5 * ce).astype(bf16)
    p2 = (0.5 * p2_ref[...]).astype(bf16)
    pb2 = (0.5 * pb2_ref[...]).astype(bf16)
    p3 = p3_ref[...].astype(bf16)
    pb3 = pb3_ref[...]

    agg_chunks = []
    for ec in range(NE // ECHUNK):
        y1 = jnp.concatenate(
            [_silu_h(base_b + ce_b[ec * ECHUNK + i:ec * ECHUNK + i + 1, :])
             for i in range(ECHUNK)], axis=0)
        y2 = _silu_h(jnp.dot(y1, p2,
                             preferred_element_type=f32).astype(bf16) + pb2)
        ge = jnp.dot(y2, p3, preferred_element_type=f32) + pb3
        contrib = ge.reshape(ECHUNK, PMAX, SCATTER) * ggm[None, :, :]
        agg_chunks.append(jnp.sum(contrib, axis=1))
    agg = jnp.concatenate(agg_chunks, axis=0)

    oo = _silu_h(0.5 * (jnp.dot(agg, o1_ref[...], preferred_element_type=f32)
                        + ob1_ref[...]))
    out_ref[0] = jnp.dot(oo, o2_ref[...], preferred_element_type=f32) + ob2_ref[...]


@jax.jit
def kernel(h, z, pos, mask, e_feat, z_emb, gw1, gb1, gw2, gb2, gw3, gb3,
           pw1, pb1, pw2, pb2, pw3, pb3, ow1, ob1, ow2, ob2):
    f32 = jnp.float32
    h = h.astype(f32)
    posm = jnp.concatenate(
        [pos.astype(f32), mask.astype(f32)[:, :, None],
         jnp.zeros((B, N, 4), f32)], axis=-1)
    zone = jax.nn.one_hot(z, ZPAD, dtype=f32)
    zemb_pad = jnp.pad(z_emb.astype(f32), ((0, ZPAD - (MAX_Z + 1)), (0, 0)))

    row = lambda v: v.reshape(1, -1)

    ii = jnp.arange(N)
    consts = jnp.stack([
        (ii[:, None] == ii[None, :]).astype(f32),
        (ii[:, None] < ii[None, :]).astype(f32),
        (ii[:, None] <= ii[None, :]).astype(f32),
        (ii[None, :] < ii[:, None]).astype(f32),
    ])

    def bspec(shape):
        nd = len(shape)
        return pl.BlockSpec((1,) + shape[1:], lambda b, _n=nd: (b,) + (0,) * (_n - 1))

    def wspec(shape):
        nd = len(shape)
        return pl.BlockSpec(shape, lambda b, _n=nd: (0,) * _n)

    batch_in = [posm, zone, h]
    weights = [consts, e_feat, zemb_pad,
               gw1, row(gb1), gw2, row(gb2), gw3, row(gb3),
               pw1, row(pb1), pw2, row(pb2), pw3, row(pb3),
               ow1, row(ob1), ow2, row(ob2)]

    out = pl.pallas_call(
        _body,
        grid=(B,),
        in_specs=[bspec(a.shape) for a in batch_in]
                 + [wspec(w.shape) for w in weights],
        out_specs=bspec((B, NE, OUT_DIM)),
        out_shape=jax.ShapeDtypeStruct((B, NE, OUT_DIM), f32),
    )(*batch_in, *weights)
    return out


# speedup vs baseline: 1.1391x; 1.0470x over previous
import jax
import jax.numpy as jnp
from jax import lax
from jax.experimental import pallas as pl

B, N, H = 8, 64, 256
NE, DE = 64, 32
RBF_DIM = 64
CUTOFF = 6.0
PMAX = 256
MAX_Z = 100
Z_EMB = 64
SCATTER = 64
GEOM_HID = 512
PAIR_HID = 256
OUT_DIM = 256
ZPAD = 128
ECHUNK = 8


def _silu_h(hx):
    return hx + hx * jnp.tanh(hx)


def _body(pos_ref, h_ref, const_ref, ef_ref, zemb_ref,
          gw1_ref, b1_ref, w2_ref, b2_ref, w3_ref, b3_ref,
          pw1_ref, pb1_ref, p2_ref, pb2_ref, p3_ref, pb3_ref,
          o1_ref, ob1_ref, o2_ref, ob2_ref, out_ref):
    f32 = jnp.float32

    posm = pos_ref[0]
    lidx = lax.broadcasted_iota(jnp.int32, (1, 8), 1)
    xyz = jnp.where(lidx < 3, posm, 0.0)
    rel = xyz - xyz[0:1, :]
    r0sq = jnp.sum(rel * rel, axis=1, keepdims=True)
    r0 = jnp.sqrt(r0sq)
    aidx = lax.broadcasted_iota(jnp.int32, (N, 1), 0).astype(f32)
    validc = jnp.where(
        (posm[:, 3:4] > 0.0) & (r0 <= CUTOFF) & (aidx > 0.0),
        1.0, 0.0).astype(f32)

    eye = const_ref[0]
    triu = const_ref[1]
    ltinc = const_ref[2]
    slow = const_ref[3]
    ones_nn = jnp.ones((N, N), f32)

    vcb = jnp.broadcast_to(validc, (N, N))
    vrow = jnp.dot(ones_nn, vcb * eye, preferred_element_type=f32)
    pv = vcb * vrow * triu

    cinc = jnp.dot(pv, ltinc, preferred_element_type=f32)
    rcnt = cinc[:, N - 1:N]
    rcb = jnp.broadcast_to(rcnt, (N, N))
    offm = jnp.dot(slow, rcb, preferred_element_type=f32)

    ones_sn = jnp.ones((PMAX, N), f32)
    O = jnp.dot(ones_sn, offm * eye, preferred_element_type=f32)
    Rc = jnp.dot(ones_sn, rcb * eye, preferred_element_type=f32)
    siota = lax.broadcasted_iota(jnp.int32, (PMAX, N), 0).astype(f32)
    Jone = jnp.where((siota >= O) & (siota < O + Rc), 1.0, 0.0).astype(f32)
    pmask = jnp.sum(Jone, axis=1, keepdims=True)

    oslot = jnp.sum(Jone * O, axis=1, keepdims=True)
    scol = lax.broadcasted_iota(jnp.int32, (PMAX, 1), 0).astype(f32)
    t = scol - oslot
    wm = cinc * pv - 1.0
    wslot = jnp.dot(Jone, wm, preferred_element_type=f32)
    Kone = jnp.where(jnp.abs(wslot - t) < 0.5, 1.0, 0.0).astype(f32) * pmask

    Vj = jnp.dot(Jone, rel, preferred_element_type=f32)
    Vk = jnp.dot(Kone, rel, preferred_element_type=f32)
    r0j = jnp.sqrt(jnp.sum(Vj * Vj, axis=1, keepdims=True))
    r0k = jnp.sqrt(jnp.sum(Vk * Vk, axis=1, keepdims=True))
    vjk = Vk - Vj
    rjk = jnp.sqrt(jnp.sum(vjk * vjk, axis=1, keepdims=True))
    dotjk = jnp.sum(Vj * Vk, axis=1, keepdims=True)
    cosang = jnp.clip(
        dotjk / (jnp.maximum(r0j, 1e-8) * jnp.maximum(r0k, 1e-8)), -1.0, 1.0)

    delta = CUTOFF / (RBF_DIM - 1)
    gamma = 1.0 / (delta * delta + 1e-12)
    centers = lax.broadcasted_iota(jnp.int32, (1, RBF_DIM), 1).astype(f32) * delta

    d_at = jnp.minimum(r0, CUTOFF) - centers
    rbf_at = jnp.exp(-gamma * d_at * d_at)
    d_jk = jnp.minimum(rjk, CUTOFF) - centers
    fjk = jnp.exp(-gamma * d_jk * d_jk)

    hb = h_ref[0]
    w1hj = gw1_ref[0:H, :]
    w1hk = gw1_ref[H:2 * H, :]
    w1fj = gw1_ref[2 * H:2 * H + RBF_DIM, :]
    w1fk = gw1_ref[2 * H + RBF_DIM:2 * H + 2 * RBF_DIM, :]
    w1jk = gw1_ref[2 * H + 2 * RBF_DIM:2 * H + 3 * RBF_DIM, :]
    w1c = gw1_ref[2 * H + 3 * RBF_DIM:2 * H + 3 * RBF_DIM + 1, :]
    atomj = (jnp.dot(hb, w1hj, preferred_element_type=f32)
             + jnp.dot(rbf_at, w1fj, preferred_element_type=f32))
    atomk = (jnp.dot(hb, w1hk, preferred_element_type=f32)
             + jnp.dot(rbf_at, w1fk, preferred_element_type=f32))
    pre = (jnp.dot(Jone, atomj, preferred_element_type=f32)
           + jnp.dot(Kone, atomk, preferred_element_type=f32)
           + jnp.dot(fjk, w1jk, preferred_element_type=f32)
           + cosang * w1c
           + b1_ref[...])
    x = _silu_h(0.5 * pre)
    x = _silu_h(0.5 * (jnp.dot(x, w2_ref[...], preferred_element_type=f32)
                       + b2_ref[...]))
    gg = jnp.dot(x, w3_ref[...], preferred_element_type=f32) + b3_ref[...]
    ggm = gg * pmask

    zcol = posm[:, 4:5]
    ziota = lax.broadcasted_iota(jnp.int32, (1, ZPAD), 1).astype(f32)
    zone = jnp.where(zcol == ziota, 1.0, 0.0).astype(f32)
    ze = jnp.dot(zone, zemb_ref[...], preferred_element_type=f32)
    pj = pw1_ref[0:Z_EMB, :]
    pk = pw1_ref[Z_EMB:2 * Z_EMB, :]
    pe = pw1_ref[2 * Z_EMB:2 * Z_EMB + DE, :]
    apj = jnp.dot(ze, pj, preferred_element_type=f32)
    apk = jnp.dot(ze, pk, preferred_element_type=f32)
    ce = jnp.dot(ef_ref[...], pe, preferred_element_type=f32)
    base = (jnp.dot(Jone, apj, preferred_element_type=f32)
            + jnp.dot(Kone, apk, preferred_element_type=f32)
            + pb1_ref[...])

    bf16 = jnp.bfloat16
    base_b = (0.5 * base).astype(bf16)
    ce_b = (0.5 * ce).astype(bf16)
    p2 = (0.5 * p2_ref[...]).astype(bf16)
    pb2 = (0.5 * pb2_ref[...]).astype(bf16)
    p3 = p3_ref[...].astype(bf16)
    pb3 = pb3_ref[...]

    agg_chunks = []
    for ec in range(NE // ECHUNK):
        y1 = jnp.concatenate(
            [_silu_h(base_b + ce_b[ec * ECHUNK + i:ec * ECHUNK + i + 1, :])
             for i in range(ECHUNK)], axis=0)
        y2 = _silu_h(jnp.dot(y1, p2,
                             preferred_element_type=f32).astype(bf16) + pb2)
        ge = jnp.dot(y2, p3, preferred_element_type=f32) + pb3
        contrib = ge.reshape(ECHUNK, PMAX, SCATTER) * ggm[None, :, :]
        agg_chunks.append(jnp.sum(contrib, axis=1))
    agg = jnp.concatenate(agg_chunks, axis=0)

    oo = _silu_h(0.5 * (jnp.dot(agg, o1_ref[...], preferred_element_type=f32)
                        + ob1_ref[...]))
    out_ref[0] = jnp.dot(oo, o2_ref[...], preferred_element_type=f32) + ob2_ref[...]


@jax.jit
def kernel(h, z, pos, mask, e_feat, z_emb, gw1, gb1, gw2, gb2, gw3, gb3,
           pw1, pb1, pw2, pb2, pw3, pb3, ow1, ob1, ow2, ob2):
    f32 = jnp.float32
    h = h.astype(f32)
    posm = jnp.concatenate(
        [pos.astype(f32), mask.astype(f32)[:, :, None],
         z.astype(f32)[:, :, None], jnp.zeros((B, N, 3), f32)],
        axis=-1)
    zemb_pad = jnp.pad(z_emb.astype(f32), ((0, ZPAD - (MAX_Z + 1)), (0, 0)))

    row = lambda v: v.reshape(1, -1)

    ii = jnp.arange(N)
    consts = jnp.stack([
        (ii[:, None] == ii[None, :]).astype(f32),
        (ii[:, None] < ii[None, :]).astype(f32),
        (ii[:, None] <= ii[None, :]).astype(f32),
        (ii[None, :] < ii[:, None]).astype(f32),
    ])

    def bspec(shape):
        nd = len(shape)
        return pl.BlockSpec((1,) + shape[1:], lambda b, _n=nd: (b,) + (0,) * (_n - 1))

    def wspec(shape):
        nd = len(shape)
        return pl.BlockSpec(shape, lambda b, _n=nd: (0,) * _n)

    batch_in = [posm, h]
    weights = [consts, e_feat, zemb_pad,
               gw1, row(gb1), gw2, row(gb2), gw3, row(gb3),
               pw1, row(pb1), pw2, row(pb2), pw3, row(pb3),
               ow1, row(ob1), ow2, row(ob2)]

    out = pl.pallas_call(
        _body,
        grid=(B,),
        in_specs=[bspec(a.shape) for a in batch_in]
                 + [wspec(w.shape) for w in weights],
        out_specs=bspec((B, NE, OUT_DIM)),
        out_shape=jax.ShapeDtypeStruct((B, NE, OUT_DIM), f32),
    )(*batch_in, *weights)
    return out


# speedup vs baseline: 1.1821x; 1.0377x over previous
import jax
import jax.numpy as jnp
from jax import lax
from jax.experimental import pallas as pl

B, N, H = 8, 64, 256
NE, DE = 64, 32
RBF_DIM = 64
CUTOFF = 6.0
PMAX = 256
MAX_Z = 100
Z_EMB = 64
SCATTER = 64
GEOM_HID = 512
PAIR_HID = 256
OUT_DIM = 256
ZPAD = 128
SUBB = 2
ECHUNK = 8


def _silu_h(hx):
    return hx + hx * jnp.tanh(hx)


def _body(pos_ref, h_ref, const_ref, ef_ref, zemb_ref,
          gw1_ref, b1_ref, w2_ref, b2_ref, w3_ref, b3_ref,
          pw1_ref, pb1_ref, p2_ref, pb2_ref, p3_ref, pb3_ref,
          o1_ref, ob1_ref, o2_ref, ob2_ref, out_ref):
    for _sb in range(SUBB):
        _one(_sb, pos_ref, h_ref, const_ref, ef_ref, zemb_ref,
             gw1_ref, b1_ref, w2_ref, b2_ref, w3_ref, b3_ref,
             pw1_ref, pb1_ref, p2_ref, pb2_ref, p3_ref, pb3_ref,
             o1_ref, ob1_ref, o2_ref, ob2_ref, out_ref)


def _one(_sb, pos_ref, h_ref, const_ref, ef_ref, zemb_ref,
         gw1_ref, b1_ref, w2_ref, b2_ref, w3_ref, b3_ref,
         pw1_ref, pb1_ref, p2_ref, pb2_ref, p3_ref, pb3_ref,
         o1_ref, ob1_ref, o2_ref, ob2_ref, out_ref):
    f32 = jnp.float32

    posm = pos_ref[_sb]
    lidx = lax.broadcasted_iota(jnp.int32, (1, 8), 1)
    xyz = jnp.where(lidx < 3, posm, 0.0)
    rel = xyz - xyz[0:1, :]
    r0sq = jnp.sum(rel * rel, axis=1, keepdims=True)
    r0 = jnp.sqrt(r0sq)
    aidx = lax.broadcasted_iota(jnp.int32, (N, 1), 0).astype(f32)
    validc = jnp.where(
        (posm[:, 3:4] > 0.0) & (r0 <= CUTOFF) & (aidx > 0.0),
        1.0, 0.0).astype(f32)

    eye = const_ref[0]
    triu = const_ref[1]
    ltinc = const_ref[2]
    slow = const_ref[3]
    ones_nn = jnp.ones((N, N), f32)

    vcb = jnp.broadcast_to(validc, (N, N))
    vrow = jnp.dot(ones_nn, vcb * eye, preferred_element_type=f32)
    pv = vcb * vrow * triu

    cinc = jnp.dot(pv, ltinc, preferred_element_type=f32)
    rcnt = cinc[:, N - 1:N]
    rcb = jnp.broadcast_to(rcnt, (N, N))
    offm = jnp.dot(slow, rcb, preferred_element_type=f32)

    ones_sn = jnp.ones((PMAX, N), f32)
    O = jnp.dot(ones_sn, offm * eye, preferred_element_type=f32)
    Rc = jnp.dot(ones_sn, rcb * eye, preferred_element_type=f32)
    siota = lax.broadcasted_iota(jnp.int32, (PMAX, N), 0).astype(f32)
    Jone = jnp.where((siota >= O) & (siota < O + Rc), 1.0, 0.0).astype(f32)
    pmask = jnp.sum(Jone, axis=1, keepdims=True)

    oslot = jnp.sum(Jone * O, axis=1, keepdims=True)
    scol = lax.broadcasted_iota(jnp.int32, (PMAX, 1), 0).astype(f32)
    t = scol - oslot
    wm = cinc * pv - 1.0
    wslot = jnp.dot(Jone, wm, preferred_element_type=f32)
    Kone = jnp.where(jnp.abs(wslot - t) < 0.5, 1.0, 0.0).astype(f32) * pmask

    Vj = jnp.dot(Jone, rel, preferred_element_type=f32)
    Vk = jnp.dot(Kone, rel, preferred_element_type=f32)
    r0j = jnp.sqrt(jnp.sum(Vj * Vj, axis=1, keepdims=True))
    r0k = jnp.sqrt(jnp.sum(Vk * Vk, axis=1, keepdims=True))
    vjk = Vk - Vj
    rjk = jnp.sqrt(jnp.sum(vjk * vjk, axis=1, keepdims=True))
    dotjk = jnp.sum(Vj * Vk, axis=1, keepdims=True)
    cosang = jnp.clip(
        dotjk / (jnp.maximum(r0j, 1e-8) * jnp.maximum(r0k, 1e-8)), -1.0, 1.0)

    delta = CUTOFF / (RBF_DIM - 1)
    gamma = 1.0 / (delta * delta + 1e-12)
    centers = lax.broadcasted_iota(jnp.int32, (1, RBF_DIM), 1).astype(f32) * delta

    d_at = jnp.minimum(r0, CUTOFF) - centers
    rbf_at = jnp.exp(-gamma * d_at * d_at)
    d_jk = jnp.minimum(rjk, CUTOFF) - centers
    fjk = jnp.exp(-gamma * d_jk * d_jk)

    hb = h_ref[_sb]
    w1hj = gw1_ref[0:H, :]
    w1hk = gw1_ref[H:2 * H, :]
    w1fj = gw1_ref[2 * H:2 * H + RBF_DIM, :]
    w1fk = gw1_ref[2 * H + RBF_DIM:2 * H + 2 * RBF_DIM, :]
    w1jk = gw1_ref[2 * H + 2 * RBF_DIM:2 * H + 3 * RBF_DIM, :]
    w1c = gw1_ref[2 * H + 3 * RBF_DIM:2 * H + 3 * RBF_DIM + 1, :]
    atomj = (jnp.dot(hb, w1hj, preferred_element_type=f32)
             + jnp.dot(rbf_at, w1fj, preferred_element_type=f32))
    atomk = (jnp.dot(hb, w1hk, preferred_element_type=f32)
             + jnp.dot(rbf_at, w1fk, preferred_element_type=f32))
    pre = (jnp.dot(Jone, atomj, preferred_element_type=f32)
           + jnp.dot(Kone, atomk, preferred_element_type=f32)
           + jnp.dot(fjk, w1jk, preferred_element_type=f32)
           + cosang * w1c
           + b1_ref[...])
    x = _silu_h(0.5 * pre)
    x = _silu_h(0.5 * (jnp.dot(x, w2_ref[...], preferred_element_type=f32)
                       + b2_ref[...]))
    gg = jnp.dot(x, w3_ref[...], preferred_element_type=f32) + b3_ref[...]
    ggm = gg * pmask

    zcol = posm[:, 4:5]
    ziota = lax.broadcasted_iota(jnp.int32, (1, ZPAD), 1).astype(f32)
    zone = jnp.where(zcol == ziota, 1.0, 0.0).astype(f32)
    ze = jnp.dot(zone, zemb_ref[...], preferred_element_type=f32)
    pj = pw1_ref[0:Z_EMB, :]
    pk = pw1_ref[Z_EMB:2 * Z_EMB, :]
    pe = pw1_ref[2 * Z_EMB:2 * Z_EMB + DE, :]
    apj = jnp.dot(ze, pj, preferred_element_type=f32)
    apk = jnp.dot(ze, pk, preferred_element_type=f32)
    ce = jnp.dot(ef_ref[...], pe, preferred_element_type=f32)
    base = (jnp.dot(Jone, apj, preferred_element_type=f32)
            + jnp.dot(Kone, apk, preferred_element_type=f32)
            + pb1_ref[...])

    bf16 = jnp.bfloat16
    base_b = (0.5 * base).astype(bf16)
    ce_b = (0.5 * ce).astype(bf16)
    p2 = (0.5 * p2_ref[...]).astype(bf16)
    pb2 = (0.5 * pb2_ref[...]).astype(bf16)
    p3 = p3_ref[...].astype(bf16)
    pb3 = pb3_ref[...]

    agg_chunks = []
    for ec in range(NE // ECHUNK):
        y1 = jnp.concatenate(
            [_silu_h(base_b + ce_b[ec * ECHUNK + i:ec * ECHUNK + i + 1, :])
             for i in range(ECHUNK)], axis=0)
        y2 = _silu_h(jnp.dot(y1, p2,
                             preferred_element_type=f32).astype(bf16) + pb2)
        ge = jnp.dot(y2, p3, preferred_element_type=f32) + pb3
        contrib = ge.reshape(ECHUNK, PMAX, SCATTER) * ggm[None, :, :]
        agg_chunks.append(jnp.sum(contrib, axis=1))
    agg = jnp.concatenate(agg_chunks, axis=0)

    oo = _silu_h(0.5 * (jnp.dot(agg, o1_ref[...], preferred_element_type=f32)
                        + ob1_ref[...]))
    out_ref[_sb] = jnp.dot(oo, o2_ref[...], preferred_element_type=f32) + ob2_ref[...]


@jax.jit
def kernel(h, z, pos, mask, e_feat, z_emb, gw1, gb1, gw2, gb2, gw3, gb3,
           pw1, pb1, pw2, pb2, pw3, pb3, ow1, ob1, ow2, ob2):
    f32 = jnp.float32
    h = h.astype(f32)
    posm = jnp.concatenate(
        [pos.astype(f32), mask.astype(f32)[:, :, None],
         z.astype(f32)[:, :, None], jnp.zeros((B, N, 3), f32)],
        axis=-1)
    zemb_pad = jnp.pad(z_emb.astype(f32), ((0, ZPAD - (MAX_Z + 1)), (0, 0)))

    row = lambda v: v.reshape(1, -1)

    ii = jnp.arange(N)
    consts = jnp.stack([
        (ii[:, None] == ii[None, :]).astype(f32),
        (ii[:, None] < ii[None, :]).astype(f32),
        (ii[:, None] <= ii[None, :]).astype(f32),
        (ii[None, :] < ii[:, None]).astype(f32),
    ])

    def bspec(shape):
        nd = len(shape)
        return pl.BlockSpec((SUBB,) + shape[1:],
                            lambda b, _n=nd: (b,) + (0,) * (_n - 1))

    def wspec(shape):
        nd = len(shape)
        return pl.BlockSpec(shape, lambda b, _n=nd: (0,) * _n)

    batch_in = [posm, h]
    weights = [consts, e_feat, zemb_pad,
               gw1, row(gb1), gw2, row(gb2), gw3, row(gb3),
               pw1, row(pb1), pw2, row(pb2), pw3, row(pb3),
               ow1, row(ob1), ow2, row(ob2)]

    out = pl.pallas_call(
        _body,
        grid=(B // SUBB,),
        in_specs=[bspec(a.shape) for a in batch_in]
                 + [wspec(w.shape) for w in weights],
        out_specs=bspec((B, NE, OUT_DIM)),
        out_shape=jax.ShapeDtypeStruct((B, NE, OUT_DIM), f32),
    )(*batch_in, *weights)
    return out


# speedup vs baseline: 1.2036x; 1.0182x over previous
import jax
import jax.numpy as jnp
from jax import lax
from jax.experimental import pallas as pl

B, N, H = 8, 64, 256
NE, DE = 64, 32
RBF_DIM = 64
CUTOFF = 6.0
PMAX = 256
MAX_Z = 100
Z_EMB = 64
SCATTER = 64
GEOM_HID = 512
PAIR_HID = 256
OUT_DIM = 256
ZPAD = 128
SUBB = 4
ECHUNK = 8


def _silu_h(hx):
    return hx + hx * jnp.tanh(hx)


def _body(pos_ref, h_ref, const_ref, ef_ref, zemb_ref,
          gw1_ref, b1_ref, w2_ref, b2_ref, w3_ref, b3_ref,
          pw1_ref, pb1_ref, p2_ref, pb2_ref, p3_ref, pb3_ref,
          o1_ref, ob1_ref, o2_ref, ob2_ref, out_ref):
    for _sb in range(SUBB):
        _one(_sb, pos_ref, h_ref, const_ref, ef_ref, zemb_ref,
             gw1_ref, b1_ref, w2_ref, b2_ref, w3_ref, b3_ref,
             pw1_ref, pb1_ref, p2_ref, pb2_ref, p3_ref, pb3_ref,
             o1_ref, ob1_ref, o2_ref, ob2_ref, out_ref)


def _one(_sb, pos_ref, h_ref, const_ref, ef_ref, zemb_ref,
         gw1_ref, b1_ref, w2_ref, b2_ref, w3_ref, b3_ref,
         pw1_ref, pb1_ref, p2_ref, pb2_ref, p3_ref, pb3_ref,
         o1_ref, ob1_ref, o2_ref, ob2_ref, out_ref):
    f32 = jnp.float32

    posm = pos_ref[_sb]
    lidx = lax.broadcasted_iota(jnp.int32, (1, 8), 1)
    xyz = jnp.where(lidx < 3, posm, 0.0)
    rel = xyz - xyz[0:1, :]
    r0sq = jnp.sum(rel * rel, axis=1, keepdims=True)
    r0 = jnp.sqrt(r0sq)
    aidx = lax.broadcasted_iota(jnp.int32, (N, 1), 0).astype(f32)
    validc = jnp.where(
        (posm[:, 3:4] > 0.0) & (r0 <= CUTOFF) & (aidx > 0.0),
        1.0, 0.0).astype(f32)

    eye = const_ref[0]
    triu = const_ref[1]
    ltinc = const_ref[2]
    slow = const_ref[3]
    ones_nn = jnp.ones((N, N), f32)

    vcb = jnp.broadcast_to(validc, (N, N))
    vrow = jnp.dot(ones_nn, vcb * eye, preferred_element_type=f32)
    pv = vcb * vrow * triu

    cinc = jnp.dot(pv, ltinc, preferred_element_type=f32)
    rcnt = cinc[:, N - 1:N]
    rcb = jnp.broadcast_to(rcnt, (N, N))
    offm = jnp.dot(slow, rcb, preferred_element_type=f32)

    ones_sn = jnp.ones((PMAX, N), f32)
    O = jnp.dot(ones_sn, offm * eye, preferred_element_type=f32)
    Rc = jnp.dot(ones_sn, rcb * eye, preferred_element_type=f32)
    siota = lax.broadcasted_iota(jnp.int32, (PMAX, N), 0).astype(f32)
    Jone = jnp.where((siota >= O) & (siota < O + Rc), 1.0, 0.0).astype(f32)
    pmask = jnp.sum(Jone, axis=1, keepdims=True)

    oslot = jnp.sum(Jone * O, axis=1, keepdims=True)
    scol = lax.broadcasted_iota(jnp.int32, (PMAX, 1), 0).astype(f32)
    t = scol - oslot
    wm = cinc * pv - 1.0
    wslot = jnp.dot(Jone, wm, preferred_element_type=f32)
    Kone = jnp.where(jnp.abs(wslot - t) < 0.5, 1.0, 0.0).astype(f32) * pmask

    Vj = jnp.dot(Jone, rel, preferred_element_type=f32)
    Vk = jnp.dot(Kone, rel, preferred_element_type=f32)
    r0j = jnp.sqrt(jnp.sum(Vj * Vj, axis=1, keepdims=True))
    r0k = jnp.sqrt(jnp.sum(Vk * Vk, axis=1, keepdims=True))
    vjk = Vk - Vj
    rjk = jnp.sqrt(jnp.sum(vjk * vjk, axis=1, keepdims=True))
    dotjk = jnp.sum(Vj * Vk, axis=1, keepdims=True)
    cosang = jnp.clip(
        dotjk / (jnp.maximum(r0j, 1e-8) * jnp.maximum(r0k, 1e-8)), -1.0, 1.0)

    delta = CUTOFF / (RBF_DIM - 1)
    gamma = 1.0 / (delta * delta + 1e-12)
    centers = lax.broadcasted_iota(jnp.int32, (1, RBF_DIM), 1).astype(f32) * delta

    d_at = jnp.minimum(r0, CUTOFF) - centers
    rbf_at = jnp.exp(-gamma * d_at * d_at)
    d_jk = jnp.minimum(rjk, CUTOFF) - centers
    fjk = jnp.exp(-gamma * d_jk * d_jk)

    hb = h_ref[_sb]
    w1hj = gw1_ref[0:H, :]
    w1hk = gw1_ref[H:2 * H, :]
    w1fj = gw1_ref[2 * H:2 * H + RBF_DIM, :]
    w1fk = gw1_ref[2 * H + RBF_DIM:2 * H + 2 * RBF_DIM, :]
    w1jk = gw1_ref[2 * H + 2 * RBF_DIM:2 * H + 3 * RBF_DIM, :]
    w1c = gw1_ref[2 * H + 3 * RBF_DIM:2 * H + 3 * RBF_DIM + 1, :]
    atomj = (jnp.dot(hb, w1hj, preferred_element_type=f32)
             + jnp.dot(rbf_at, w1fj, preferred_element_type=f32))
    atomk = (jnp.dot(hb, w1hk, preferred_element_type=f32)
             + jnp.dot(rbf_at, w1fk, preferred_element_type=f32))
    pre = (jnp.dot(Jone, atomj, preferred_element_type=f32)
           + jnp.dot(Kone, atomk, preferred_element_type=f32)
           + jnp.dot(fjk, w1jk, preferred_element_type=f32)
           + cosang * w1c
           + b1_ref[...])
    x = _silu_h(0.5 * pre)
    x = _silu_h(0.5 * (jnp.dot(x, w2_ref[...], preferred_element_type=f32)
                       + b2_ref[...]))
    gg = jnp.dot(x, w3_ref[...], preferred_element_type=f32) + b3_ref[...]
    ggm = gg * pmask

    zcol = posm[:, 4:5]
    ziota = lax.broadcasted_iota(jnp.int32, (1, ZPAD), 1).astype(f32)
    zone = jnp.where(zcol == ziota, 1.0, 0.0).astype(f32)
    ze = jnp.dot(zone, zemb_ref[...], preferred_element_type=f32)
    pj = pw1_ref[0:Z_EMB, :]
    pk = pw1_ref[Z_EMB:2 * Z_EMB, :]
    pe = pw1_ref[2 * Z_EMB:2 * Z_EMB + DE, :]
    apj = jnp.dot(ze, pj, preferred_element_type=f32)
    apk = jnp.dot(ze, pk, preferred_element_type=f32)
    ce = jnp.dot(ef_ref[...], pe, preferred_element_type=f32)
    base = (jnp.dot(Jone, apj, preferred_element_type=f32)
            + jnp.dot(Kone, apk, preferred_element_type=f32)
            + pb1_ref[...])

    bf16 = jnp.bfloat16
    base_b = (0.5 * base).astype(bf16)
    ce_b = (0.5 * ce).astype(bf16)
    p2 = (0.5 * p2_ref[...]).astype(bf16)
    pb2 = (0.5 * pb2_ref[...]).astype(bf16)
    p3 = p3_ref[...].astype(bf16)
    pb3 = pb3_ref[...]

    agg_chunks = []
    for ec in range(NE // ECHUNK):
        y1 = jnp.concatenate(
            [_silu_h(base_b + ce_b[ec * ECHUNK + i:ec * ECHUNK + i + 1, :])
             for i in range(ECHUNK)], axis=0)
        y2 = _silu_h(jnp.dot(y1, p2,
                             preferred_element_type=f32).astype(bf16) + pb2)
        ge = jnp.dot(y2, p3, preferred_element_type=f32) + pb3
        contrib = ge.reshape(ECHUNK, PMAX, SCATTER) * ggm[None, :, :]
        agg_chunks.append(jnp.sum(contrib, axis=1))
    agg = jnp.concatenate(agg_chunks, axis=0)

    oo = _silu_h(0.5 * (jnp.dot(agg, o1_ref[...], preferred_element_type=f32)
                        + ob1_ref[...]))
    out_ref[_sb] = jnp.dot(oo, o2_ref[...], preferred_element_type=f32) + ob2_ref[...]


@jax.jit
def kernel(h, z, pos, mask, e_feat, z_emb, gw1, gb1, gw2, gb2, gw3, gb3,
           pw1, pb1, pw2, pb2, pw3, pb3, ow1, ob1, ow2, ob2):
    f32 = jnp.float32
    h = h.astype(f32)
    posm = jnp.concatenate(
        [pos.astype(f32), mask.astype(f32)[:, :, None],
         z.astype(f32)[:, :, None], jnp.zeros((B, N, 3), f32)],
        axis=-1)
    zemb_pad = jnp.pad(z_emb.astype(f32), ((0, ZPAD - (MAX_Z + 1)), (0, 0)))

    row = lambda v: v.reshape(1, -1)

    ii = jnp.arange(N)
    consts = jnp.stack([
        (ii[:, None] == ii[None, :]).astype(f32),
        (ii[:, None] < ii[None, :]).astype(f32),
        (ii[:, None] <= ii[None, :]).astype(f32),
        (ii[None, :] < ii[:, None]).astype(f32),
    ])

    def bspec(shape):
        nd = len(shape)
        return pl.BlockSpec((SUBB,) + shape[1:],
                            lambda b, _n=nd: (b,) + (0,) * (_n - 1))

    def wspec(shape):
        nd = len(shape)
        return pl.BlockSpec(shape, lambda b, _n=nd: (0,) * _n)

    batch_in = [posm, h]
    weights = [consts, e_feat, zemb_pad,
               gw1, row(gb1), gw2, row(gb2), gw3, row(gb3),
               pw1, row(pb1), pw2, row(pb2), pw3, row(pb3),
               ow1, row(ob1), ow2, row(ob2)]

    out = pl.pallas_call(
        _body,
        grid=(B // SUBB,),
        in_specs=[bspec(a.shape) for a in batch_in]
                 + [wspec(w.shape) for w in weights],
        out_specs=bspec((B, NE, OUT_DIM)),
        out_shape=jax.ShapeDtypeStruct((B, NE, OUT_DIM), f32),
    )(*batch_in, *weights)
    return out
